# baseline (speedup 1.0000x reference)
import functools

import jax
import jax.numpy as jnp
from jax import lax
from jax.experimental import pallas as pl
from jax.experimental.pallas import tpu as pltpu
from jax.experimental.pallas import tpu_sc as plsc

BATCH = 16384
NC = 2
NS = 16
NW = NC * NS
B_PER_W = BATCH // NW
CHUNK = 128
NCHUNK = B_PER_W // CHUNK

D_USER, D_GENDER, D_GENRE, D_ARTIST = 64, 16, 32, 64

_sc_mesh = plsc.VectorSubcoreMesh(core_axis_name="c", subcore_axis_name="s")


@functools.partial(
    pl.kernel,
    out_type=(
        jax.ShapeDtypeStruct((BATCH, D_USER), jnp.float32),
        jax.ShapeDtypeStruct((BATCH, D_GENDER), jnp.float32),
        jax.ShapeDtypeStruct((BATCH, D_GENRE), jnp.float32),
        jax.ShapeDtypeStruct((BATCH, D_ARTIST), jnp.float32),
    ),
    mesh=_sc_mesh,
    scratch_types=(
        pltpu.VMEM((NCHUNK, CHUNK), jnp.int32),
        pltpu.VMEM((NCHUNK, CHUNK), jnp.int32),
        pltpu.VMEM((NCHUNK, CHUNK), jnp.int32),
        pltpu.VMEM((NCHUNK, CHUNK), jnp.int32),
        pltpu.VMEM((B_PER_W, D_USER), jnp.float32),
        pltpu.VMEM((B_PER_W, D_GENDER), jnp.float32),
        pltpu.VMEM((B_PER_W, D_GENRE), jnp.float32),
        pltpu.VMEM((B_PER_W, D_ARTIST), jnp.float32),
        pltpu.SemaphoreType.DMA,
        pltpu.SemaphoreType.DMA,
        pltpu.SemaphoreType.DMA,
        pltpu.SemaphoreType.DMA,
    ),
    compiler_params=pltpu.CompilerParams(use_tc_tiling_on_sc=False),
)
def _sc_gather(uid_hbm, gid_hbm, gnr_hbm, aid_hbm,
               ut_hbm, gt_hbm, gnt_hbm, at_hbm,
               out_u, out_g, out_gn, out_a,
               uidx, gidx, gnidx, aidx,
               urows, grows, gnrows, arows,
               usem, gsem, gnsem, asem):
    wid = lax.axis_index("s") * NC + lax.axis_index("c")
    row0 = wid * NCHUNK
    base = wid * B_PER_W

    pltpu.sync_copy(uid_hbm.at[pl.ds(row0, NCHUNK)], uidx)
    pltpu.sync_copy(gid_hbm.at[pl.ds(row0, NCHUNK)], gidx)
    pltpu.sync_copy(gnr_hbm.at[pl.ds(row0, NCHUNK)], gnidx)
    pltpu.sync_copy(aid_hbm.at[pl.ds(row0, NCHUNK)], aidx)

    copies = []
    for j in range(NCHUNK):
        sl = pl.ds(j * CHUNK, CHUNK)
        copies.append(pltpu.async_copy(ut_hbm.at[uidx.at[j]], urows.at[sl], usem))
        copies.append(pltpu.async_copy(gt_hbm.at[gidx.at[j]], grows.at[sl], gsem))
        copies.append(pltpu.async_copy(gnt_hbm.at[gnidx.at[j]], gnrows.at[sl], gnsem))
        copies.append(pltpu.async_copy(at_hbm.at[aidx.at[j]], arows.at[sl], asem))
    for c in copies:
        c.wait()

    out_sl = pl.ds(base, B_PER_W)
    pltpu.sync_copy(urows, out_u.at[out_sl])
    pltpu.sync_copy(grows, out_g.at[out_sl])
    pltpu.sync_copy(gnrows, out_gn.at[out_sl])
    pltpu.sync_copy(arows, out_a.at[out_sl])


def _mlp_body(x_ref, w1_ref, b1_ref, w2_ref, b2_ref, o_ref):
    h = lax.dot_general(x_ref[:], w1_ref[:], (((1,), (1,)), ((), ())),
                        preferred_element_type=jnp.float32)
    h = jnp.maximum(h + b1_ref[:], 0.0)
    o = lax.dot_general(h, w2_ref[:], (((1,), (1,)), ((), ())),
                        preferred_element_type=jnp.float32)
    o_ref[:] = o + b2_ref[:]


_MLP_BLK = 1024


@jax.jit
def _mlp(audio_features, W1, b1, W2, b2):
    grid = (BATCH // _MLP_BLK,)
    return pl.pallas_call(
        _mlp_body,
        grid=grid,
        in_specs=[
            pl.BlockSpec((_MLP_BLK, 128), lambda i: (i, 0)),
            pl.BlockSpec((256, 128), lambda i: (0, 0)),
            pl.BlockSpec((1, 256), lambda i: (0, 0)),
            pl.BlockSpec((128, 256), lambda i: (0, 0)),
            pl.BlockSpec((1, 128), lambda i: (0, 0)),
        ],
        out_specs=pl.BlockSpec((_MLP_BLK, 128), lambda i: (i, 0)),
        out_shape=jax.ShapeDtypeStruct((BATCH, 128), jnp.float32),
    )(audio_features, W1, b1.reshape(1, 256), W2, b2.reshape(1, 128))


@jax.jit
def kernel(user_ids, genders, genres, artist_ids, audio_features,
           user_table, gender_table, genre_table, artist_table,
           W1, b1, W2, b2):
    uid2 = user_ids.reshape(BATCH // CHUNK, CHUNK).astype(jnp.int32)
    gid2 = genders.reshape(BATCH // CHUNK, CHUNK).astype(jnp.int32)
    gnr2 = genres.reshape(BATCH // CHUNK, CHUNK).astype(jnp.int32)
    aid2 = artist_ids.reshape(BATCH // CHUNK, CHUNK).astype(jnp.int32)

    out_u, out_g, out_gn, out_a = _sc_gather(
        uid2, gid2, gnr2, aid2,
        user_table, gender_table, genre_table, artist_table)

    audio_emb = _mlp(audio_features, W1, b1, W2, b2)
    return (out_u, out_g, out_gn, out_a, audio_emb)


# speedup vs baseline: 1.6573x; 1.6573x over previous
import functools

import jax
import jax.numpy as jnp
from jax import lax
from jax.experimental import pallas as pl
from jax.experimental.pallas import tpu as pltpu
from jax.experimental.pallas import tpu_sc as plsc

BATCH = 16384
NC = 2
NS = 16
NW = NC * NS
B_PER_W = BATCH // NW
CHUNK = 128
NCHUNK = B_PER_W // CHUNK

D_USER, D_GENDER, D_GENRE, D_ARTIST = 64, 16, 32, 64
N_GENDER, N_GENRE = 4, 1000

_sc_mesh = plsc.VectorSubcoreMesh(core_axis_name="c", subcore_axis_name="s")


@functools.partial(
    pl.kernel,
    out_type=(
        jax.ShapeDtypeStruct((BATCH, D_USER), jnp.float32),
        jax.ShapeDtypeStruct((BATCH, D_GENDER), jnp.float32),
        jax.ShapeDtypeStruct((BATCH, D_GENRE), jnp.float32),
        jax.ShapeDtypeStruct((BATCH, D_ARTIST), jnp.float32),
    ),
    mesh=_sc_mesh,
    scratch_types=(
        pltpu.VMEM((B_PER_W,), jnp.int32),
        pltpu.VMEM((B_PER_W,), jnp.int32),
        pltpu.VMEM((B_PER_W,), jnp.int32),
        pltpu.VMEM((B_PER_W,), jnp.int32),
        pltpu.VMEM((CHUNK, D_USER), jnp.float32),
        pltpu.VMEM((CHUNK, D_GENDER), jnp.float32),
        pltpu.VMEM((CHUNK, D_GENRE), jnp.float32),
        pltpu.VMEM((CHUNK, D_ARTIST), jnp.float32),
        pltpu.VMEM((N_GENDER, D_GENDER), jnp.float32),
        pltpu.SemaphoreType.DMA,
        pltpu.SemaphoreType.DMA,
        pltpu.SemaphoreType.DMA,
        pltpu.SemaphoreType.DMA,
    ),
)
def _sc_gather(uid_hbm, gid_hbm, gnr_hbm, aid_hbm,
               ut_hbm, gt_hbm, gnt_hbm, at_hbm,
               out_u, out_g, out_gn, out_a,
               uidx, gidx, gnidx, aidx,
               urows, grows, gnrows, arows,
               gtab,
               usem, asem, gnsem, ssem):
    wid = lax.axis_index("s") * NC + lax.axis_index("c")
    base = wid * B_PER_W

    in_sl = pl.ds(base, B_PER_W)
    pltpu.sync_copy(uid_hbm.at[in_sl], uidx)
    pltpu.sync_copy(gnr_hbm.at[in_sl], gnidx)
    pltpu.sync_copy(aid_hbm.at[in_sl], aidx)
    pltpu.sync_copy(gid_hbm.at[in_sl], gidx)
    pltpu.sync_copy(gt_hbm, gtab)

    def chunk_body(c, carry):
        def fire_block(b, carry2):
            i0 = c * CHUNK + b * 16
            uv = uidx[pl.ds(i0, 16)]
            av = aidx[pl.ds(i0, 16)]
            gv = gnidx[pl.ds(i0, 16)]
            for l in range(16):
                i = b * 16 + l
                pltpu.async_copy(ut_hbm.at[uv[l]], urows.at[i], usem)
                pltpu.async_copy(at_hbm.at[av[l]], arows.at[i], asem)
                pltpu.async_copy(gnt_hbm.at[gv[l]], gnrows.at[i], gnsem)
            return carry2

        lax.fori_loop(0, CHUNK // 16, fire_block, 0)

        def sel_block(b, carry2):
            gv = gidx[pl.ds(c * CHUNK + b * 16, 16)]
            for l in range(16):
                grows[b * 16 + l, :] = gtab[gv[l], :]
            return carry2

        lax.fori_loop(0, CHUNK // 16, sel_block, 0)

        pltpu.make_async_copy(out_u.at[pl.ds(0, CHUNK)], urows, usem).wait()
        pltpu.make_async_copy(out_a.at[pl.ds(0, CHUNK)], arows, asem).wait()
        pltpu.make_async_copy(out_gn.at[pl.ds(0, CHUNK)], gnrows, gnsem).wait()

        out_sl = pl.ds(base + c * CHUNK, CHUNK)
        pltpu.sync_copy(urows, out_u.at[out_sl])
        pltpu.sync_copy(grows, out_g.at[out_sl])
        pltpu.sync_copy(gnrows, out_gn.at[out_sl])
        pltpu.sync_copy(arows, out_a.at[out_sl])
        return carry

    lax.fori_loop(0, NCHUNK, chunk_body, 0)


def _mlp_body(x_ref, w1_ref, b1_ref, w2_ref, b2_ref, o_ref):
    h = lax.dot_general(x_ref[:], w1_ref[:], (((1,), (1,)), ((), ())),
                        preferred_element_type=jnp.float32)
    h = jnp.maximum(h + b1_ref[:], 0.0)
    o = lax.dot_general(h, w2_ref[:], (((1,), (1,)), ((), ())),
                        preferred_element_type=jnp.float32)
    o_ref[:] = o + b2_ref[:]


_MLP_BLK = 1024


@jax.jit
def _mlp(audio_features, W1, b1, W2, b2):
    grid = (BATCH // _MLP_BLK,)
    return pl.pallas_call(
        _mlp_body,
        grid=grid,
        in_specs=[
            pl.BlockSpec((_MLP_BLK, 128), lambda i: (i, 0)),
            pl.BlockSpec((256, 128), lambda i: (0, 0)),
            pl.BlockSpec((1, 256), lambda i: (0, 0)),
            pl.BlockSpec((128, 256), lambda i: (0, 0)),
            pl.BlockSpec((1, 128), lambda i: (0, 0)),
        ],
        out_specs=pl.BlockSpec((_MLP_BLK, 128), lambda i: (i, 0)),
        out_shape=jax.ShapeDtypeStruct((BATCH, 128), jnp.float32),
    )(audio_features, W1, b1.reshape(1, 256), W2, b2.reshape(1, 128))


@jax.jit
def kernel(user_ids, genders, genres, artist_ids, audio_features,
           user_table, gender_table, genre_table, artist_table,
           W1, b1, W2, b2):
    out_u, out_g, out_gn, out_a = _sc_gather(
        user_ids.astype(jnp.int32), genders.astype(jnp.int32),
        genres.astype(jnp.int32), artist_ids.astype(jnp.int32),
        user_table, gender_table, genre_table, artist_table)

    audio_emb = _mlp(audio_features, W1, b1, W2, b2)
    return (out_u, out_g, out_gn, out_a, audio_emb)
